# Initial kernel scaffold; baseline (speedup 1.0000x reference)
import jax
import jax.numpy as jnp
from jax.experimental import pallas as pl


def kernel(x, edge_index, edge_attr, params):
    raise NotImplementedError("write your pallas kernel here")


# speedup vs baseline: 2.4036x; 2.4036x over previous
import functools

import jax
import jax.numpy as jnp
from jax import lax
from jax.experimental import pallas as pl
from jax.experimental.pallas import tpu as pltpu
from jax.experimental.pallas import tpu_sc as plsc

H = 64
NC = 2
NS = 16
NW = NC * NS
GK = 400
SK = 400
BE = 4000


def _tc_pre(x, Wn, bn, Wi, Wj):
    N = x.shape[0]

    def body(x_ref, wn_ref, bn_ref, wi_ref, wj_ref, h_ref, a_ref, b_ref):
        h = jnp.dot(x_ref[...], wn_ref[...], preferred_element_type=jnp.float32)
        h = h + bn_ref[...]
        h_ref[...] = h
        a_ref[...] = jnp.dot(h, wi_ref[...], preferred_element_type=jnp.float32)
        b_ref[...] = jnp.dot(h, wj_ref[...], preferred_element_type=jnp.float32)

    out = pl.pallas_call(
        body,
        out_shape=[jax.ShapeDtypeStruct((N, H), jnp.float32)] * 3,
    )(x, Wn, bn.reshape(1, H), Wi, Wj)
    return out


def _sc_gather(A, B, dst, src):
    E = dst.shape[0]
    epw = E // NW
    nch = epw // GK
    mesh = plsc.VectorSubcoreMesh(core_axis_name="c", subcore_axis_name="s")

    @functools.partial(
        pl.kernel,
        out_type=[jax.ShapeDtypeStruct((E, H), jnp.float32),
                  jax.ShapeDtypeStruct((E, H), jnp.float32)],
        mesh=mesh,
        compiler_params=pltpu.CompilerParams(use_tc_tiling_on_sc=False),
        scratch_types=[
            pltpu.VMEM((GK,), jnp.int32),
            pltpu.VMEM((GK,), jnp.int32),
            pltpu.VMEM((GK, H), jnp.float32),
            pltpu.VMEM((GK, H), jnp.float32),
            pltpu.SemaphoreType.DMA,
            pltpu.SemaphoreType.DMA,
        ],
    )
    def k(a_hbm, b_hbm, dst_hbm, src_hbm, ar_hbm, br_hbm,
          idxd_v, idxs_v, a_v, b_v, sem1, sem2):
        wid = lax.axis_index("s") * NC + lax.axis_index("c")
        base0 = wid * epw

        def chunk(cix, carry):
            base = base0 + cix * GK
            pltpu.sync_copy(dst_hbm.at[pl.ds(base, GK)], idxd_v)
            pltpu.sync_copy(src_hbm.at[pl.ds(base, GK)], idxs_v)
            cp1 = pltpu.async_copy(a_hbm.at[idxd_v], a_v, sem1)
            cp2 = pltpu.async_copy(b_hbm.at[idxs_v], b_v, sem2)
            cp1.wait()
            cp2.wait()
            pltpu.sync_copy(a_v, ar_hbm.at[pl.ds(base, GK)])
            pltpu.sync_copy(b_v, br_hbm.at[pl.ds(base, GK)])
            return carry

        lax.fori_loop(0, nch, chunk, 0)

    return k(A, B, dst, src)


def _tc_edge(ar, br, e, W1e, b1, W2, b2):
    E, De = e.shape

    def body(ar_ref, br_ref, e_ref, w1_ref, b1_ref, w2_ref, b2_ref, m_ref):
        c = jnp.dot(e_ref[...], w1_ref[...], preferred_element_type=jnp.float32)
        p = jnp.maximum(ar_ref[...] + br_ref[...] + c + b1_ref[...], 0.0)
        m = jnp.dot(p, w2_ref[...], preferred_element_type=jnp.float32)
        m_ref[...] = jnp.maximum(m + b2_ref[...], 0.0)

    return pl.pallas_call(
        body,
        grid=(E // BE,),
        in_specs=[
            pl.BlockSpec((BE, H), lambda i: (i, 0)),
            pl.BlockSpec((BE, H), lambda i: (i, 0)),
            pl.BlockSpec((BE, De), lambda i: (i, 0)),
            pl.BlockSpec((De, H), lambda i: (0, 0)),
            pl.BlockSpec((1, H), lambda i: (0, 0)),
            pl.BlockSpec((H, H), lambda i: (0, 0)),
            pl.BlockSpec((1, H), lambda i: (0, 0)),
        ],
        out_specs=pl.BlockSpec((BE, H), lambda i: (i, 0)),
        out_shape=jax.ShapeDtypeStruct((E, H), jnp.float32),
    )(ar, br, e, W1e, b1.reshape(1, H), W2, b2.reshape(1, H))


def _sc_scatter(M, dst, zeros_tile, N):
    E = dst.shape[0]
    epw = E // NW
    nch = epw // SK
    npt = N // NS
    mesh = plsc.VectorSubcoreMesh(core_axis_name="c", subcore_axis_name="s")

    @functools.partial(
        pl.kernel,
        out_type=jax.ShapeDtypeStruct((NC, N, H), jnp.float32),
        mesh=mesh,
        compiler_params=pltpu.CompilerParams(use_tc_tiling_on_sc=False),
        scratch_types=[
            pltpu.VMEM((SK,), jnp.int32),
            pltpu.VMEM((SK, H), jnp.float32),
            pltpu.VMEM_SHARED((N, H), jnp.float32),
        ],
    )
    def k(m_hbm, dst_hbm, z_hbm, out_hbm, idx_v, m_v, acc_sh):
        c = lax.axis_index("c")
        s = lax.axis_index("s")
        wid = s * NC + c
        pltpu.sync_copy(z_hbm, acc_sh.at[pl.ds(s * npt, npt)])
        plsc.subcore_barrier()

        def chunk(cix, carry):
            base = wid * epw + cix * SK
            pltpu.sync_copy(dst_hbm.at[pl.ds(base, SK)], idx_v)
            pltpu.sync_copy(m_hbm.at[pl.ds(base, SK)], m_v)
            pltpu.sync_copy(m_v, acc_sh.at[idx_v], add=True)
            return carry

        lax.fori_loop(0, nch, chunk, 0)
        plsc.subcore_barrier()
        pltpu.sync_copy(acc_sh.at[pl.ds(s * npt, npt)],
                        out_hbm.at[c, pl.ds(s * npt, npt)])

    return k(M, dst, zeros_tile)


def _tc_update(h, accs, W1h, W1a, b1, W2, b2, Wi, Wj):
    N = h.shape[0]

    def body(h_ref, p0_ref, p1_ref, w1h_ref, w1a_ref, b1_ref, w2_ref, b2_ref,
             wi_ref, wj_ref, h_out, a_out, b_out):
        aggr = p0_ref[...] + p1_ref[...]
        u = jnp.dot(h_ref[...], w1h_ref[...], preferred_element_type=jnp.float32)
        u = u + jnp.dot(aggr, w1a_ref[...], preferred_element_type=jnp.float32)
        u = jnp.maximum(u + b1_ref[...], 0.0)
        u = jnp.dot(u, w2_ref[...], preferred_element_type=jnp.float32)
        u = jnp.maximum(u + b2_ref[...], 0.0)
        hn = u + h_ref[...]
        h_out[...] = hn
        a_out[...] = jnp.dot(hn, wi_ref[...], preferred_element_type=jnp.float32)
        b_out[...] = jnp.dot(hn, wj_ref[...], preferred_element_type=jnp.float32)

    return pl.pallas_call(
        body,
        out_shape=[jax.ShapeDtypeStruct((N, H), jnp.float32)] * 3,
    )(h, accs[0], accs[1], W1h, W1a, b1.reshape(1, H), W2, b2.reshape(1, H),
      Wi, Wj)


def _tc_final(h, accs, W1h, W1a, b1, W2, b2, tW1, tb1, tW2r, tb2):
    N = h.shape[0]
    Hh = tW1.shape[1]

    def body(h_ref, p0_ref, p1_ref, w1h_ref, w1a_ref, b1_ref, w2_ref, b2_ref,
             tw1_ref, tb1_ref, tw2_ref, tb2_ref, h_out, tok_out):
        aggr = p0_ref[...] + p1_ref[...]
        u = jnp.dot(h_ref[...], w1h_ref[...], preferred_element_type=jnp.float32)
        u = u + jnp.dot(aggr, w1a_ref[...], preferred_element_type=jnp.float32)
        u = jnp.maximum(u + b1_ref[...], 0.0)
        u = jnp.dot(u, w2_ref[...], preferred_element_type=jnp.float32)
        u = jnp.maximum(u + b2_ref[...], 0.0)
        hn = u + h_ref[...]
        h_out[...] = hn
        t = jnp.dot(hn, tw1_ref[...], preferred_element_type=jnp.float32)
        t = jnp.maximum(t + tb1_ref[...], 0.0)
        tok_out[...] = jnp.sum(t * tw2_ref[...], axis=1) + tb2_ref[0, 0]

    return pl.pallas_call(
        body,
        out_shape=[jax.ShapeDtypeStruct((N, H), jnp.float32),
                   jax.ShapeDtypeStruct((N,), jnp.float32)],
    )(h, accs[0], accs[1], W1h, W1a, b1.reshape(1, H), W2, b2.reshape(1, H),
      tW1, tb1.reshape(1, Hh), tW2r, tb2.reshape(1, 1))


def kernel(x, edge_index, edge_attr, params):
    p = params
    N = x.shape[0]
    src = edge_index[0]
    dst = edge_index[1]
    zeros_tile = jnp.zeros((N // NS, H), jnp.float32)

    h, A, B = _tc_pre(x, p['node_proj_W'], p['node_proj_b'],
                      p['l0_msg_W1'][:H], p['l0_msg_W1'][H:2 * H])
    num_layers = 3
    for i in range(num_layers):
        W1 = p[f'l{i}_msg_W1']
        W1e = p['edge_proj_W'] @ W1[2 * H:]
        b1e = p['edge_proj_b'] @ W1[2 * H:] + p[f'l{i}_msg_b1']
        ar, br = _sc_gather(A, B, dst, src)
        M = _tc_edge(ar, br, edge_attr, W1e, b1e,
                     p[f'l{i}_msg_W2'], p[f'l{i}_msg_b2'])
        accs = _sc_scatter(M, dst, zeros_tile, N)
        upW1 = p[f'l{i}_up_W1']
        if i < num_layers - 1:
            Wn1 = p[f'l{i + 1}_msg_W1']
            h, A, B = _tc_update(h, accs, upW1[:H], upW1[H:],
                                 p[f'l{i}_up_b1'], p[f'l{i}_up_W2'],
                                 p[f'l{i}_up_b2'], Wn1[:H], Wn1[H:2 * H])
        else:
            h, tok = _tc_final(h, accs, upW1[:H], upW1[H:],
                               p[f'l{i}_up_b1'], p[f'l{i}_up_W2'],
                               p[f'l{i}_up_b2'], p['tok_W1'], p['tok_b1'],
                               p['tok_W2'].reshape(1, H // 2), p['tok_b2'])
    return tok, h


# speedup vs baseline: 3.2667x; 1.3591x over previous
import functools

import jax
import jax.numpy as jnp
from jax import lax
from jax.experimental import pallas as pl
from jax.experimental.pallas import tpu as pltpu
from jax.experimental.pallas import tpu_sc as plsc

H = 64
NC = 2
NS = 16
NW = NC * NS
GK = 200
SK = 200
BE = 4000


def _tc_pre(x, Wn, bn, Wi, Wj):
    N = x.shape[0]

    def body(x_ref, wn_ref, bn_ref, wi_ref, wj_ref, h_ref, a_ref, b_ref):
        h = jnp.dot(x_ref[...], wn_ref[...], preferred_element_type=jnp.float32)
        h = h + bn_ref[...]
        h_ref[...] = h
        a_ref[...] = jnp.dot(h, wi_ref[...], preferred_element_type=jnp.float32)
        b_ref[...] = jnp.dot(h, wj_ref[...], preferred_element_type=jnp.float32)

    out = pl.pallas_call(
        body,
        out_shape=[jax.ShapeDtypeStruct((N, H), jnp.float32)] * 3,
    )(x, Wn, bn.reshape(1, H), Wi, Wj)
    return out


def _sc_gather(A, B, dst, src):
    E = dst.shape[0]
    epw = E // NW
    nch = epw // GK
    mesh = plsc.VectorSubcoreMesh(core_axis_name="c", subcore_axis_name="s")

    @functools.partial(
        pl.kernel,
        out_type=jax.ShapeDtypeStruct((E, H), jnp.float32),
        mesh=mesh,
        compiler_params=pltpu.CompilerParams(use_tc_tiling_on_sc=False),
        scratch_types=[
            pltpu.VMEM((epw,), jnp.int32),
            pltpu.VMEM((epw,), jnp.int32),
            pltpu.VMEM((2, GK, H), jnp.float32),
            pltpu.VMEM((2, GK, H), jnp.float32),
            pltpu.SemaphoreType.DMA,
            pltpu.SemaphoreType.DMA,
        ],
    )
    def k(a_hbm, b_hbm, dst_hbm, src_hbm, g_hbm,
          idxd_all, idxs_all, a_v, b_v, sem_a, sem_b):
        wid = lax.axis_index("s") * NC + lax.axis_index("c")
        base0 = wid * epw
        pltpu.sync_copy(dst_hbm.at[pl.ds(base0, epw)], idxd_all)
        pltpu.sync_copy(src_hbm.at[pl.ds(base0, epw)], idxs_all)
        pltpu.async_copy(a_hbm.at[idxd_all.at[pl.ds(0, GK)]], a_v.at[0], sem_a)
        pltpu.async_copy(b_hbm.at[idxs_all.at[pl.ds(0, GK)]], b_v.at[0], sem_b)

        def step(j, carry):
            for p in range(2):
                g = 2 * j + p
                pltpu.make_async_copy(
                    a_hbm.at[pl.ds(0, GK)], a_v.at[p], sem_a).wait()
                pltpu.make_async_copy(
                    b_hbm.at[pl.ds(0, GK)], b_v.at[p], sem_b).wait()

                @pl.when(g + 1 < nch)
                def _():
                    off = (g + 1) * GK
                    pltpu.async_copy(a_hbm.at[idxd_all.at[pl.ds(off, GK)]],
                                     a_v.at[1 - p], sem_a)
                    pltpu.async_copy(b_hbm.at[idxs_all.at[pl.ds(off, GK)]],
                                     b_v.at[1 - p], sem_b)

                def row(r, c2):
                    for t in range(H // 16):
                        sl = pl.ds(t * 16, 16)
                        a_v[p, r, sl] = a_v[p, r, sl] + b_v[p, r, sl]
                    return c2

                lax.fori_loop(0, GK, row, 0)
                pltpu.sync_copy(a_v.at[p],
                                g_hbm.at[pl.ds(base0 + g * GK, GK)])
            return carry

        lax.fori_loop(0, nch // 2, step, 0)

    return k(A, B, dst, src)


def _tc_edge(g, e, W1e, b1, W2, b2):
    E, De = e.shape

    def body(g_ref, e_ref, w1_ref, b1_ref, w2_ref, b2_ref, m_ref):
        c = jnp.dot(e_ref[...], w1_ref[...], preferred_element_type=jnp.float32)
        p = jnp.maximum(g_ref[...] + c + b1_ref[...], 0.0)
        m = jnp.dot(p, w2_ref[...], preferred_element_type=jnp.float32)
        m_ref[...] = jnp.maximum(m + b2_ref[...], 0.0)

    return pl.pallas_call(
        body,
        grid=(E // BE,),
        in_specs=[
            pl.BlockSpec((BE, H), lambda i: (i, 0)),
            pl.BlockSpec((BE, De), lambda i: (i, 0)),
            pl.BlockSpec((De, H), lambda i: (0, 0)),
            pl.BlockSpec((1, H), lambda i: (0, 0)),
            pl.BlockSpec((H, H), lambda i: (0, 0)),
            pl.BlockSpec((1, H), lambda i: (0, 0)),
        ],
        out_specs=pl.BlockSpec((BE, H), lambda i: (i, 0)),
        out_shape=jax.ShapeDtypeStruct((E, H), jnp.float32),
    )(g, e, W1e, b1.reshape(1, H), W2, b2.reshape(1, H))


def _sc_scatter(M, dst, zeros_tile, N):
    E = dst.shape[0]
    epw = E // NW
    nch = epw // SK
    npt = N // NS
    mesh = plsc.VectorSubcoreMesh(core_axis_name="c", subcore_axis_name="s")

    @functools.partial(
        pl.kernel,
        out_type=jax.ShapeDtypeStruct((NC, N, H), jnp.float32),
        mesh=mesh,
        compiler_params=pltpu.CompilerParams(use_tc_tiling_on_sc=False),
        scratch_types=[
            pltpu.VMEM((2, SK), jnp.int32),
            pltpu.VMEM((2, SK, H), jnp.float32),
            pltpu.VMEM_SHARED((N, H), jnp.float32),
            pltpu.SemaphoreType.DMA,
            pltpu.SemaphoreType.DMA,
        ],
    )
    def k(m_hbm, dst_hbm, z_hbm, out_hbm, idx_v, m_v, acc_sh, sem_i, sem_m):
        c = lax.axis_index("c")
        s = lax.axis_index("s")
        wid = s * NC + c
        base0 = wid * epw
        pltpu.sync_copy(z_hbm, acc_sh.at[pl.ds(s * npt, npt)])
        plsc.subcore_barrier()
        pltpu.async_copy(dst_hbm.at[pl.ds(base0, SK)], idx_v.at[0], sem_i)
        pltpu.async_copy(m_hbm.at[pl.ds(base0, SK)], m_v.at[0], sem_m)

        def step(j, carry):
            for p in range(2):
                g = 2 * j + p
                pltpu.make_async_copy(
                    dst_hbm.at[pl.ds(0, SK)], idx_v.at[p], sem_i).wait()
                pltpu.make_async_copy(
                    m_hbm.at[pl.ds(0, SK)], m_v.at[p], sem_m).wait()

                @pl.when(g + 1 < nch)
                def _():
                    off = base0 + (g + 1) * SK
                    pltpu.async_copy(dst_hbm.at[pl.ds(off, SK)],
                                     idx_v.at[1 - p], sem_i)
                    pltpu.async_copy(m_hbm.at[pl.ds(off, SK)],
                                     m_v.at[1 - p], sem_m)

                pltpu.sync_copy(m_v.at[p], acc_sh.at[idx_v.at[p]], add=True)
            return carry

        lax.fori_loop(0, nch // 2, step, 0)
        plsc.subcore_barrier()
        pltpu.sync_copy(acc_sh.at[pl.ds(s * npt, npt)],
                        out_hbm.at[c, pl.ds(s * npt, npt)])

    return k(M, dst, zeros_tile)


def _tc_update(h, accs, W1h, W1a, b1, W2, b2, Wi, Wj):
    N = h.shape[0]

    def body(h_ref, p0_ref, p1_ref, w1h_ref, w1a_ref, b1_ref, w2_ref, b2_ref,
             wi_ref, wj_ref, h_out, a_out, b_out):
        aggr = p0_ref[...] + p1_ref[...]
        u = jnp.dot(h_ref[...], w1h_ref[...], preferred_element_type=jnp.float32)
        u = u + jnp.dot(aggr, w1a_ref[...], preferred_element_type=jnp.float32)
        u = jnp.maximum(u + b1_ref[...], 0.0)
        u = jnp.dot(u, w2_ref[...], preferred_element_type=jnp.float32)
        u = jnp.maximum(u + b2_ref[...], 0.0)
        hn = u + h_ref[...]
        h_out[...] = hn
        a_out[...] = jnp.dot(hn, wi_ref[...], preferred_element_type=jnp.float32)
        b_out[...] = jnp.dot(hn, wj_ref[...], preferred_element_type=jnp.float32)

    return pl.pallas_call(
        body,
        out_shape=[jax.ShapeDtypeStruct((N, H), jnp.float32)] * 3,
    )(h, accs[0], accs[1], W1h, W1a, b1.reshape(1, H), W2, b2.reshape(1, H),
      Wi, Wj)


def _tc_final(h, accs, W1h, W1a, b1, W2, b2, tW1, tb1, tW2r, tb2):
    N = h.shape[0]
    Hh = tW1.shape[1]

    def body(h_ref, p0_ref, p1_ref, w1h_ref, w1a_ref, b1_ref, w2_ref, b2_ref,
             tw1_ref, tb1_ref, tw2_ref, tb2_ref, h_out, tok_out):
        aggr = p0_ref[...] + p1_ref[...]
        u = jnp.dot(h_ref[...], w1h_ref[...], preferred_element_type=jnp.float32)
        u = u + jnp.dot(aggr, w1a_ref[...], preferred_element_type=jnp.float32)
        u = jnp.maximum(u + b1_ref[...], 0.0)
        u = jnp.dot(u, w2_ref[...], preferred_element_type=jnp.float32)
        u = jnp.maximum(u + b2_ref[...], 0.0)
        hn = u + h_ref[...]
        h_out[...] = hn
        t = jnp.dot(hn, tw1_ref[...], preferred_element_type=jnp.float32)
        t = jnp.maximum(t + tb1_ref[...], 0.0)
        tok_out[...] = jnp.sum(t * tw2_ref[...], axis=1) + tb2_ref[0, 0]

    return pl.pallas_call(
        body,
        out_shape=[jax.ShapeDtypeStruct((N, H), jnp.float32),
                   jax.ShapeDtypeStruct((N,), jnp.float32)],
    )(h, accs[0], accs[1], W1h, W1a, b1.reshape(1, H), W2, b2.reshape(1, H),
      tW1, tb1.reshape(1, Hh), tW2r, tb2.reshape(1, 1))


def kernel(x, edge_index, edge_attr, params):
    p = params
    N = x.shape[0]
    src = edge_index[0]
    dst = edge_index[1]
    zeros_tile = jnp.zeros((N // NS, H), jnp.float32)

    h, A, B = _tc_pre(x, p['node_proj_W'], p['node_proj_b'],
                      p['l0_msg_W1'][:H], p['l0_msg_W1'][H:2 * H])
    num_layers = 3
    for i in range(num_layers):
        W1 = p[f'l{i}_msg_W1']
        W1e = p['edge_proj_W'] @ W1[2 * H:]
        b1e = p['edge_proj_b'] @ W1[2 * H:] + p[f'l{i}_msg_b1']
        g = _sc_gather(A, B, dst, src)
        M = _tc_edge(g, edge_attr, W1e, b1e,
                     p[f'l{i}_msg_W2'], p[f'l{i}_msg_b2'])
        accs = _sc_scatter(M, dst, zeros_tile, N)
        upW1 = p[f'l{i}_up_W1']
        if i < num_layers - 1:
            Wn1 = p[f'l{i + 1}_msg_W1']
            h, A, B = _tc_update(h, accs, upW1[:H], upW1[H:],
                                 p[f'l{i}_up_b1'], p[f'l{i}_up_W2'],
                                 p[f'l{i}_up_b2'], Wn1[:H], Wn1[H:2 * H])
        else:
            h, tok = _tc_final(h, accs, upW1[:H], upW1[H:],
                               p[f'l{i}_up_b1'], p[f'l{i}_up_W2'],
                               p[f'l{i}_up_b2'], p['tok_W1'], p['tok_b1'],
                               p['tok_W2'].reshape(1, H // 2), p['tok_b2'])
    return tok, h


# speedup vs baseline: 5.7598x; 1.7632x over previous
import functools

import jax
import jax.numpy as jnp
from jax import lax
from jax.experimental import pallas as pl
from jax.experimental.pallas import tpu as pltpu
from jax.experimental.pallas import tpu_sc as plsc

H = 64
NC = 2
NS = 16
NW = NC * NS
GK = 200
SK = 200
BE = 4000


def _tc_pre(x, Wn, bn, Wi, Wj):
    N = x.shape[0]

    def body(x_ref, wn_ref, bn_ref, wi_ref, wj_ref, h_ref, a_ref, b_ref):
        h = jnp.dot(x_ref[...], wn_ref[...], preferred_element_type=jnp.float32)
        h = h + bn_ref[...]
        h_ref[...] = h
        a_ref[...] = jnp.dot(h, wi_ref[...], preferred_element_type=jnp.float32)
        b_ref[...] = jnp.dot(h, wj_ref[...], preferred_element_type=jnp.float32)

    out = pl.pallas_call(
        body,
        out_shape=[jax.ShapeDtypeStruct((N, H), jnp.float32)] * 3,
    )(x, Wn, bn.reshape(1, H), Wi, Wj)
    return out


def _sc_gather(A, B, dst, src):
    E = dst.shape[0]
    epw = E // NW
    nch = epw // GK
    mesh = plsc.VectorSubcoreMesh(core_axis_name="c", subcore_axis_name="s")

    @functools.partial(
        pl.kernel,
        out_type=jax.ShapeDtypeStruct((E // 2, 2 * H), jnp.float32),
        mesh=mesh,
        compiler_params=pltpu.CompilerParams(use_tc_tiling_on_sc=False),
        scratch_types=[
            pltpu.VMEM((epw,), jnp.int32),
            pltpu.VMEM((epw,), jnp.int32),
            pltpu.VMEM((2, GK, H), jnp.float32),
            pltpu.VMEM((2, GK, H), jnp.float32),
            pltpu.VMEM((2, GK // 2, 2 * H), jnp.float32),
            pltpu.SemaphoreType.DMA,
            pltpu.SemaphoreType.DMA,
        ],
    )
    def k(a_hbm, b_hbm, dst_hbm, src_hbm, g_hbm,
          idxd_all, idxs_all, a_v, b_v, o_v, sem_a, sem_b):
        wid = lax.axis_index("s") * NC + lax.axis_index("c")
        base0 = wid * epw
        pltpu.sync_copy(dst_hbm.at[pl.ds(base0, epw)], idxd_all)
        pltpu.sync_copy(src_hbm.at[pl.ds(base0, epw)], idxs_all)
        pltpu.async_copy(a_hbm.at[idxd_all.at[pl.ds(0, GK)]], a_v.at[0], sem_a)
        pltpu.async_copy(b_hbm.at[idxs_all.at[pl.ds(0, GK)]], b_v.at[0], sem_b)

        def step(j, carry):
            for p in range(2):
                g = 2 * j + p
                pltpu.make_async_copy(
                    a_hbm.at[pl.ds(0, GK)], a_v.at[p], sem_a).wait()
                pltpu.make_async_copy(
                    b_hbm.at[pl.ds(0, GK)], b_v.at[p], sem_b).wait()

                @pl.when(g + 1 < nch)
                def _():
                    off = (g + 1) * GK
                    pltpu.async_copy(a_hbm.at[idxd_all.at[pl.ds(off, GK)]],
                                     a_v.at[1 - p], sem_a)
                    pltpu.async_copy(b_hbm.at[idxs_all.at[pl.ds(off, GK)]],
                                     b_v.at[1 - p], sem_b)

                def row(rp, c2):
                    for half in range(2):
                        for t in range(H // 16):
                            sl = pl.ds(t * 16, 16)
                            ol = pl.ds(half * H + t * 16, 16)
                            o_v[p, rp, ol] = (a_v[p, 2 * rp + half, sl]
                                              + b_v[p, 2 * rp + half, sl])
                    return c2

                lax.fori_loop(0, GK // 2, row, 0)
                pltpu.sync_copy(o_v.at[p],
                                g_hbm.at[pl.ds((base0 + g * GK) // 2, GK // 2)])
            return carry

        lax.fori_loop(0, nch // 2, step, 0)

    return k(A, B, dst, src)


def _tc_edge(g, e2, W1e, b1, W2, b2):
    E2 = e2.shape[0]
    De2 = e2.shape[1]
    BEP = BE // 2
    W1e2 = jnp.kron(jnp.eye(2, dtype=jnp.float32), W1e)
    W2p = jnp.kron(jnp.eye(2, dtype=jnp.float32), W2)
    b1p = jnp.tile(b1, 2).reshape(1, 2 * H)
    b2p = jnp.tile(b2, 2).reshape(1, 2 * H)

    def body(g_ref, e_ref, w1_ref, b1_ref, w2_ref, b2_ref, m_ref):
        c = jnp.dot(e_ref[...], w1_ref[...], preferred_element_type=jnp.float32)
        p = jnp.maximum(g_ref[...] + c + b1_ref[...], 0.0)
        m = jnp.dot(p, w2_ref[...], preferred_element_type=jnp.float32)
        m_ref[...] = jnp.maximum(m + b2_ref[...], 0.0)

    return pl.pallas_call(
        body,
        grid=(E2 // BEP,),
        in_specs=[
            pl.BlockSpec((BEP, 2 * H), lambda i: (i, 0)),
            pl.BlockSpec((BEP, De2), lambda i: (i, 0)),
            pl.BlockSpec((De2, 2 * H), lambda i: (0, 0)),
            pl.BlockSpec((1, 2 * H), lambda i: (0, 0)),
            pl.BlockSpec((2 * H, 2 * H), lambda i: (0, 0)),
            pl.BlockSpec((1, 2 * H), lambda i: (0, 0)),
        ],
        out_specs=pl.BlockSpec((BEP, 2 * H), lambda i: (i, 0)),
        out_shape=jax.ShapeDtypeStruct((E2, 2 * H), jnp.float32),
    )(g, e2, W1e2, b1p, W2p, b2p)


def _sc_scatter(M, dst, zeros_tile, N):
    E = dst.shape[0]
    epw = E // NW
    nch = epw // SK
    npt = N // NS
    mesh = plsc.VectorSubcoreMesh(core_axis_name="c", subcore_axis_name="s")

    @functools.partial(
        pl.kernel,
        out_type=jax.ShapeDtypeStruct((NC, N, H), jnp.float32),
        mesh=mesh,
        compiler_params=pltpu.CompilerParams(use_tc_tiling_on_sc=False),
        scratch_types=[
            pltpu.VMEM((2, SK), jnp.int32),
            pltpu.VMEM((2, SK // 2, 2 * H), jnp.float32),
            pltpu.VMEM((SK, H), jnp.float32),
            pltpu.VMEM_SHARED((N, H), jnp.float32),
            pltpu.SemaphoreType.DMA,
            pltpu.SemaphoreType.DMA,
        ],
    )
    def k(m_hbm, dst_hbm, z_hbm, out_hbm, idx_v, m_v, m64_v, acc_sh,
          sem_i, sem_m):
        c = lax.axis_index("c")
        s = lax.axis_index("s")
        wid = s * NC + c
        base0 = wid * epw
        pltpu.sync_copy(z_hbm, acc_sh.at[pl.ds(s * npt, npt)])
        plsc.subcore_barrier()
        pltpu.async_copy(dst_hbm.at[pl.ds(base0, SK)], idx_v.at[0], sem_i)
        pltpu.async_copy(m_hbm.at[pl.ds(base0 // 2, SK // 2)], m_v.at[0],
                         sem_m)

        def step(j, carry):
            for p in range(2):
                g = 2 * j + p
                pltpu.make_async_copy(
                    dst_hbm.at[pl.ds(0, SK)], idx_v.at[p], sem_i).wait()
                pltpu.make_async_copy(
                    m_hbm.at[pl.ds(0, SK // 2)], m_v.at[p], sem_m).wait()

                @pl.when(g + 1 < nch)
                def _():
                    off = base0 + (g + 1) * SK
                    pltpu.async_copy(dst_hbm.at[pl.ds(off, SK)],
                                     idx_v.at[1 - p], sem_i)
                    pltpu.async_copy(m_hbm.at[pl.ds(off // 2, SK // 2)],
                                     m_v.at[1 - p], sem_m)

                def row(rp, c2):
                    for half in range(2):
                        for t in range(H // 16):
                            sl = pl.ds(half * H + t * 16, 16)
                            ol = pl.ds(t * 16, 16)
                            m64_v[2 * rp + half, ol] = m_v[p, rp, sl]
                    return c2

                lax.fori_loop(0, SK // 2, row, 0)
                pltpu.sync_copy(m64_v, acc_sh.at[idx_v.at[p]], add=True)
            return carry

        lax.fori_loop(0, nch // 2, step, 0)
        plsc.subcore_barrier()
        pltpu.sync_copy(acc_sh.at[pl.ds(s * npt, npt)],
                        out_hbm.at[c, pl.ds(s * npt, npt)])

    return k(M, dst, zeros_tile)


def _tc_update(h, accs, W1h, W1a, b1, W2, b2, Wi, Wj):
    N = h.shape[0]

    def body(h_ref, p0_ref, p1_ref, w1h_ref, w1a_ref, b1_ref, w2_ref, b2_ref,
             wi_ref, wj_ref, h_out, a_out, b_out):
        aggr = p0_ref[...] + p1_ref[...]
        u = jnp.dot(h_ref[...], w1h_ref[...], preferred_element_type=jnp.float32)
        u = u + jnp.dot(aggr, w1a_ref[...], preferred_element_type=jnp.float32)
        u = jnp.maximum(u + b1_ref[...], 0.0)
        u = jnp.dot(u, w2_ref[...], preferred_element_type=jnp.float32)
        u = jnp.maximum(u + b2_ref[...], 0.0)
        hn = u + h_ref[...]
        h_out[...] = hn
        a_out[...] = jnp.dot(hn, wi_ref[...], preferred_element_type=jnp.float32)
        b_out[...] = jnp.dot(hn, wj_ref[...], preferred_element_type=jnp.float32)

    return pl.pallas_call(
        body,
        out_shape=[jax.ShapeDtypeStruct((N, H), jnp.float32)] * 3,
    )(h, accs[0], accs[1], W1h, W1a, b1.reshape(1, H), W2, b2.reshape(1, H),
      Wi, Wj)


def _tc_final(h, accs, W1h, W1a, b1, W2, b2, tW1, tb1, tW2r, tb2):
    N = h.shape[0]
    Hh = tW1.shape[1]

    def body(h_ref, p0_ref, p1_ref, w1h_ref, w1a_ref, b1_ref, w2_ref, b2_ref,
             tw1_ref, tb1_ref, tw2_ref, tb2_ref, h_out, tok_out):
        aggr = p0_ref[...] + p1_ref[...]
        u = jnp.dot(h_ref[...], w1h_ref[...], preferred_element_type=jnp.float32)
        u = u + jnp.dot(aggr, w1a_ref[...], preferred_element_type=jnp.float32)
        u = jnp.maximum(u + b1_ref[...], 0.0)
        u = jnp.dot(u, w2_ref[...], preferred_element_type=jnp.float32)
        u = jnp.maximum(u + b2_ref[...], 0.0)
        hn = u + h_ref[...]
        h_out[...] = hn
        t = jnp.dot(hn, tw1_ref[...], preferred_element_type=jnp.float32)
        t = jnp.maximum(t + tb1_ref[...], 0.0)
        tok_out[...] = jnp.sum(t * tw2_ref[...], axis=1) + tb2_ref[0, 0]

    return pl.pallas_call(
        body,
        out_shape=[jax.ShapeDtypeStruct((N, H), jnp.float32),
                   jax.ShapeDtypeStruct((N,), jnp.float32)],
    )(h, accs[0], accs[1], W1h, W1a, b1.reshape(1, H), W2, b2.reshape(1, H),
      tW1, tb1.reshape(1, Hh), tW2r, tb2.reshape(1, 1))


def kernel(x, edge_index, edge_attr, params):
    p = params
    N = x.shape[0]
    E = edge_index.shape[1]
    src = edge_index[0]
    dst = edge_index[1]
    e2 = edge_attr.reshape(E // 2, 2 * edge_attr.shape[1])
    zeros_tile = jnp.zeros((N // NS, H), jnp.float32)

    h, A, B = _tc_pre(x, p['node_proj_W'], p['node_proj_b'],
                      p['l0_msg_W1'][:H], p['l0_msg_W1'][H:2 * H])
    num_layers = 3
    for i in range(num_layers):
        W1 = p[f'l{i}_msg_W1']
        W1e = p['edge_proj_W'] @ W1[2 * H:]
        b1e = p['edge_proj_b'] @ W1[2 * H:] + p[f'l{i}_msg_b1']
        g = _sc_gather(A, B, dst, src)
        M = _tc_edge(g, e2, W1e, b1e,
                     p[f'l{i}_msg_W2'], p[f'l{i}_msg_b2'])
        accs = _sc_scatter(M, dst, zeros_tile, N)
        upW1 = p[f'l{i}_up_W1']
        if i < num_layers - 1:
            Wn1 = p[f'l{i + 1}_msg_W1']
            h, A, B = _tc_update(h, accs, upW1[:H], upW1[H:],
                                 p[f'l{i}_up_b1'], p[f'l{i}_up_W2'],
                                 p[f'l{i}_up_b2'], Wn1[:H], Wn1[H:2 * H])
        else:
            h, tok = _tc_final(h, accs, upW1[:H], upW1[H:],
                               p[f'l{i}_up_b1'], p[f'l{i}_up_W2'],
                               p[f'l{i}_up_b2'], p['tok_W1'], p['tok_b1'],
                               p['tok_W2'].reshape(1, H // 2), p['tok_b2'])
    return tok, h


# speedup vs baseline: 6.1571x; 1.0690x over previous
import functools

import jax
import jax.numpy as jnp
from jax import lax
from jax.experimental import pallas as pl
from jax.experimental.pallas import tpu as pltpu
from jax.experimental.pallas import tpu_sc as plsc

H = 64
NC = 2
NS = 16
NW = NC * NS
GK = 200
SK = 200
BEP = 2000
NPART = 2


def _tc_pre(x, Wn, bn, Wi, Wj):
    N = x.shape[0]

    def body(x_ref, wn_ref, bn_ref, wi_ref, wj_ref, h_ref, a_ref, b_ref):
        h = jnp.dot(x_ref[...], wn_ref[...], preferred_element_type=jnp.float32)
        h = h + bn_ref[...]
        h_ref[...] = h
        a_ref[...] = jnp.dot(h, wi_ref[...], preferred_element_type=jnp.float32)
        b_ref[...] = jnp.dot(h, wj_ref[...], preferred_element_type=jnp.float32)

    out = pl.pallas_call(
        body,
        out_shape=[jax.ShapeDtypeStruct((N, H), jnp.float32)] * 3,
    )(x, Wn, bn.reshape(1, H), Wi, Wj)
    return out


def _sc_gather(A, B, dst, src, off, ne):
    epw = ne // NW
    nch = epw // GK
    mesh = plsc.VectorSubcoreMesh(core_axis_name="c", subcore_axis_name="s")

    @functools.partial(
        pl.kernel,
        out_type=jax.ShapeDtypeStruct((ne // 2, 2 * H), jnp.float32),
        mesh=mesh,
        compiler_params=pltpu.CompilerParams(use_tc_tiling_on_sc=False),
        scratch_types=[
            pltpu.VMEM((epw,), jnp.int32),
            pltpu.VMEM((epw,), jnp.int32),
            pltpu.VMEM((2, GK, H), jnp.float32),
            pltpu.VMEM((2, GK, H), jnp.float32),
            pltpu.VMEM((2, GK // 2, 2 * H), jnp.float32),
            pltpu.SemaphoreType.DMA,
            pltpu.SemaphoreType.DMA,
        ],
    )
    def k(a_hbm, b_hbm, dst_hbm, src_hbm, g_hbm,
          idxd_all, idxs_all, a_v, b_v, o_v, sem_a, sem_b):
        wid = lax.axis_index("s") * NC + lax.axis_index("c")
        l0 = wid * epw
        pltpu.sync_copy(dst_hbm.at[pl.ds(off + l0, epw)], idxd_all)
        pltpu.sync_copy(src_hbm.at[pl.ds(off + l0, epw)], idxs_all)
        pltpu.async_copy(a_hbm.at[idxd_all.at[pl.ds(0, GK)]], a_v.at[0], sem_a)
        pltpu.async_copy(b_hbm.at[idxs_all.at[pl.ds(0, GK)]], b_v.at[0], sem_b)

        def step(j, carry):
            for p in range(2):
                g = 2 * j + p

                @pl.when(g < nch)
                def _():
                    pltpu.make_async_copy(
                        a_hbm.at[pl.ds(0, GK)], a_v.at[p], sem_a).wait()
                    pltpu.make_async_copy(
                        b_hbm.at[pl.ds(0, GK)], b_v.at[p], sem_b).wait()

                    @pl.when(g + 1 < nch)
                    def _():
                        o = (g + 1) * GK
                        pltpu.async_copy(a_hbm.at[idxd_all.at[pl.ds(o, GK)]],
                                         a_v.at[1 - p], sem_a)
                        pltpu.async_copy(b_hbm.at[idxs_all.at[pl.ds(o, GK)]],
                                         b_v.at[1 - p], sem_b)

                    def row(rp, c2):
                        for half in range(2):
                            for t in range(H // 16):
                                sl = pl.ds(t * 16, 16)
                                ol = pl.ds(half * H + t * 16, 16)
                                o_v[p, rp, ol] = (a_v[p, 2 * rp + half, sl]
                                                  + b_v[p, 2 * rp + half, sl])
                        return c2

                    lax.fori_loop(0, GK // 2, row, 0)
                    pltpu.sync_copy(
                        o_v.at[p],
                        g_hbm.at[pl.ds((l0 + g * GK) // 2, GK // 2)])
            return carry

        lax.fori_loop(0, (nch + 1) // 2, step, 0)

    return k(A, B, dst, src)


def _tc_edge(g, e2, W1e, b1, W2, b2, pair_off):
    ne2 = g.shape[0]
    De2 = e2.shape[1]
    blk_off = pair_off // BEP
    W1e2 = jnp.kron(jnp.eye(2, dtype=jnp.float32), W1e)
    W2p = jnp.kron(jnp.eye(2, dtype=jnp.float32), W2)
    b1p = jnp.tile(b1, 2).reshape(1, 2 * H)
    b2p = jnp.tile(b2, 2).reshape(1, 2 * H)

    def body(g_ref, e_ref, w1_ref, b1_ref, w2_ref, b2_ref, m_ref):
        c = jnp.dot(e_ref[...], w1_ref[...], preferred_element_type=jnp.float32)
        p = jnp.maximum(g_ref[...] + c + b1_ref[...], 0.0)
        m = jnp.dot(p, w2_ref[...], preferred_element_type=jnp.float32)
        m_ref[...] = jnp.maximum(m + b2_ref[...], 0.0)

    return pl.pallas_call(
        body,
        grid=(ne2 // BEP,),
        in_specs=[
            pl.BlockSpec((BEP, 2 * H), lambda i: (i, 0)),
            pl.BlockSpec((BEP, De2), lambda i: (i + blk_off, 0)),
            pl.BlockSpec((De2, 2 * H), lambda i: (0, 0)),
            pl.BlockSpec((1, 2 * H), lambda i: (0, 0)),
            pl.BlockSpec((2 * H, 2 * H), lambda i: (0, 0)),
            pl.BlockSpec((1, 2 * H), lambda i: (0, 0)),
        ],
        out_specs=pl.BlockSpec((BEP, 2 * H), lambda i: (i, 0)),
        out_shape=jax.ShapeDtypeStruct((ne2, 2 * H), jnp.float32),
    )(g, e2, W1e2, b1p, W2p, b2p)


def _sc_scatter(M, dst, zeros_tile, N, off, ne):
    epw = ne // NW
    nch = epw // SK
    npt = N // NS
    mesh = plsc.VectorSubcoreMesh(core_axis_name="c", subcore_axis_name="s")

    @functools.partial(
        pl.kernel,
        out_type=jax.ShapeDtypeStruct((NC, N, H), jnp.float32),
        mesh=mesh,
        compiler_params=pltpu.CompilerParams(use_tc_tiling_on_sc=False),
        scratch_types=[
            pltpu.VMEM((2, SK), jnp.int32),
            pltpu.VMEM((2, SK // 2, 2 * H), jnp.float32),
            pltpu.VMEM((SK, H), jnp.float32),
            pltpu.VMEM_SHARED((N, H), jnp.float32),
            pltpu.SemaphoreType.DMA,
            pltpu.SemaphoreType.DMA,
        ],
    )
    def k(m_hbm, dst_hbm, z_hbm, out_hbm, idx_v, m_v, m64_v, acc_sh,
          sem_i, sem_m):
        c = lax.axis_index("c")
        s = lax.axis_index("s")
        wid = s * NC + c
        l0 = wid * epw
        pltpu.sync_copy(z_hbm, acc_sh.at[pl.ds(s * npt, npt)])
        plsc.subcore_barrier()
        pltpu.async_copy(dst_hbm.at[pl.ds(off + l0, SK)], idx_v.at[0], sem_i)
        pltpu.async_copy(m_hbm.at[pl.ds(l0 // 2, SK // 2)], m_v.at[0], sem_m)

        def step(j, carry):
            for p in range(2):
                g = 2 * j + p

                @pl.when(g < nch)
                def _():
                    pltpu.make_async_copy(
                        dst_hbm.at[pl.ds(0, SK)], idx_v.at[p], sem_i).wait()
                    pltpu.make_async_copy(
                        m_hbm.at[pl.ds(0, SK // 2)], m_v.at[p], sem_m).wait()

                    @pl.when(g + 1 < nch)
                    def _():
                        o = l0 + (g + 1) * SK
                        pltpu.async_copy(dst_hbm.at[pl.ds(off + o, SK)],
                                         idx_v.at[1 - p], sem_i)
                        pltpu.async_copy(m_hbm.at[pl.ds(o // 2, SK // 2)],
                                         m_v.at[1 - p], sem_m)

                    def row(rp, c2):
                        for half in range(2):
                            for t in range(H // 16):
                                sl = pl.ds(half * H + t * 16, 16)
                                ol = pl.ds(t * 16, 16)
                                m64_v[2 * rp + half, ol] = m_v[p, rp, sl]
                        return c2

                    lax.fori_loop(0, SK // 2, row, 0)
                    pltpu.sync_copy(m64_v, acc_sh.at[idx_v.at[p]], add=True)
            return carry

        lax.fori_loop(0, (nch + 1) // 2, step, 0)
        plsc.subcore_barrier()
        pltpu.sync_copy(acc_sh.at[pl.ds(s * npt, npt)],
                        out_hbm.at[c, pl.ds(s * npt, npt)])

    return k(M, dst, zeros_tile)


def _tc_update(h, accs, W1h, W1a, b1, W2, b2, Wi, Wj):
    N = h.shape[0]

    def body(h_ref, p0_ref, p1_ref, p2_ref, p3_ref, w1h_ref, w1a_ref, b1_ref,
             w2_ref, b2_ref, wi_ref, wj_ref, h_out, a_out, b_out):
        aggr = ((p0_ref[...] + p1_ref[...]) + (p2_ref[...] + p3_ref[...]))
        u = jnp.dot(h_ref[...], w1h_ref[...], preferred_element_type=jnp.float32)
        u = u + jnp.dot(aggr, w1a_ref[...], preferred_element_type=jnp.float32)
        u = jnp.maximum(u + b1_ref[...], 0.0)
        u = jnp.dot(u, w2_ref[...], preferred_element_type=jnp.float32)
        u = jnp.maximum(u + b2_ref[...], 0.0)
        hn = u + h_ref[...]
        h_out[...] = hn
        a_out[...] = jnp.dot(hn, wi_ref[...], preferred_element_type=jnp.float32)
        b_out[...] = jnp.dot(hn, wj_ref[...], preferred_element_type=jnp.float32)

    return pl.pallas_call(
        body,
        out_shape=[jax.ShapeDtypeStruct((N, H), jnp.float32)] * 3,
    )(h, accs[0][0], accs[0][1], accs[1][0], accs[1][1], W1h, W1a,
      b1.reshape(1, H), W2, b2.reshape(1, H), Wi, Wj)


def _tc_final(h, accs, W1h, W1a, b1, W2, b2, tW1, tb1, tW2r, tb2):
    N = h.shape[0]
    Hh = tW1.shape[1]

    def body(h_ref, p0_ref, p1_ref, p2_ref, p3_ref, w1h_ref, w1a_ref, b1_ref,
             w2_ref, b2_ref, tw1_ref, tb1_ref, tw2_ref, tb2_ref,
             h_out, tok_out):
        aggr = ((p0_ref[...] + p1_ref[...]) + (p2_ref[...] + p3_ref[...]))
        u = jnp.dot(h_ref[...], w1h_ref[...], preferred_element_type=jnp.float32)
        u = u + jnp.dot(aggr, w1a_ref[...], preferred_element_type=jnp.float32)
        u = jnp.maximum(u + b1_ref[...], 0.0)
        u = jnp.dot(u, w2_ref[...], preferred_element_type=jnp.float32)
        u = jnp.maximum(u + b2_ref[...], 0.0)
        hn = u + h_ref[...]
        h_out[...] = hn
        t = jnp.dot(hn, tw1_ref[...], preferred_element_type=jnp.float32)
        t = jnp.maximum(t + tb1_ref[...], 0.0)
        tok_out[...] = jnp.sum(t * tw2_ref[...], axis=1) + tb2_ref[0, 0]

    return pl.pallas_call(
        body,
        out_shape=[jax.ShapeDtypeStruct((N, H), jnp.float32),
                   jax.ShapeDtypeStruct((N,), jnp.float32)],
    )(h, accs[0][0], accs[0][1], accs[1][0], accs[1][1], W1h, W1a,
      b1.reshape(1, H), W2, b2.reshape(1, H),
      tW1, tb1.reshape(1, Hh), tW2r, tb2.reshape(1, 1))


def kernel(x, edge_index, edge_attr, params):
    p = params
    N = x.shape[0]
    E = edge_index.shape[1]
    ne = E // NPART
    src = edge_index[0]
    dst = edge_index[1]
    e2 = edge_attr.reshape(E // 2, 2 * edge_attr.shape[1])
    zeros_tile = jnp.zeros((N // NS, H), jnp.float32)

    h, A, B = _tc_pre(x, p['node_proj_W'], p['node_proj_b'],
                      p['l0_msg_W1'][:H], p['l0_msg_W1'][H:2 * H])
    num_layers = 3
    for i in range(num_layers):
        W1 = p[f'l{i}_msg_W1']
        W1e = p['edge_proj_W'] @ W1[2 * H:]
        b1e = p['edge_proj_b'] @ W1[2 * H:] + p[f'l{i}_msg_b1']
        gs = [None] * NPART
        Ms = [None] * NPART
        accs = [None] * NPART
        for k in range(NPART):
            gs[k] = _sc_gather(A, B, dst, src, k * ne, ne)
            if k > 0:
                Ms[k - 1] = _tc_edge(gs[k - 1], e2, W1e, b1e,
                                     p[f'l{i}_msg_W2'], p[f'l{i}_msg_b2'],
                                     (k - 1) * ne // 2)
        Ms[NPART - 1] = _tc_edge(gs[NPART - 1], e2, W1e, b1e,
                                 p[f'l{i}_msg_W2'], p[f'l{i}_msg_b2'],
                                 (NPART - 1) * ne // 2)
        for k in range(NPART):
            accs[k] = _sc_scatter(Ms[k], dst, zeros_tile, N, k * ne, ne)
        upW1 = p[f'l{i}_up_W1']
        if i < num_layers - 1:
            Wn1 = p[f'l{i + 1}_msg_W1']
            h, A, B = _tc_update(h, accs, upW1[:H], upW1[H:],
                                 p[f'l{i}_up_b1'], p[f'l{i}_up_W2'],
                                 p[f'l{i}_up_b2'], Wn1[:H], Wn1[H:2 * H])
        else:
            h, tok = _tc_final(h, accs, upW1[:H], upW1[H:],
                               p[f'l{i}_up_b1'], p[f'l{i}_up_W2'],
                               p[f'l{i}_up_b2'], p['tok_W1'], p['tok_b1'],
                               p['tok_W2'].reshape(1, H // 2), p['tok_b2'])
    return tok, h


# speedup vs baseline: 6.4763x; 1.0518x over previous
import functools

import jax
import jax.numpy as jnp
from jax import lax
from jax.experimental import pallas as pl
from jax.experimental.pallas import tpu as pltpu
from jax.experimental.pallas import tpu_sc as plsc

H = 64
NC = 2
NS = 16
NW = NC * NS
GK = 200
SK = 200
BEP = 2000
NPART = 2


def _tc_pre(x, Wn, bn, Wi, Wj):
    N = x.shape[0]

    def body(x_ref, wn_ref, bn_ref, wi_ref, wj_ref, h_ref, a_ref, b_ref):
        h = jnp.dot(x_ref[...], wn_ref[...], preferred_element_type=jnp.float32)
        h = h + bn_ref[...]
        h_ref[...] = h
        a_ref[...] = jnp.dot(h, wi_ref[...], preferred_element_type=jnp.float32)
        b_ref[...] = jnp.dot(h, wj_ref[...], preferred_element_type=jnp.float32)

    out = pl.pallas_call(
        body,
        out_shape=[jax.ShapeDtypeStruct((N, H), jnp.float32)] * 3,
    )(x, Wn, bn.reshape(1, H), Wi, Wj)
    return out


def _sc_gather(A, B, edge_index, off, ne, N):
    epw = ne // NW
    nch = epw // GK
    npt = N // NS
    mesh = plsc.VectorSubcoreMesh(core_axis_name="c", subcore_axis_name="s")

    @functools.partial(
        pl.kernel,
        out_type=jax.ShapeDtypeStruct((ne // 2, 2 * H), jnp.float32),
        mesh=mesh,
        compiler_params=pltpu.CompilerParams(use_tc_tiling_on_sc=False),
        scratch_types=[
            pltpu.VMEM((epw,), jnp.int32),
            pltpu.VMEM((epw,), jnp.int32),
            pltpu.VMEM((2, GK, H), jnp.float32),
            pltpu.VMEM((2, GK, H), jnp.float32),
            pltpu.VMEM((2, GK // 2, 2 * H), jnp.float32),
            pltpu.VMEM_SHARED((N, H), jnp.float32),
            pltpu.SemaphoreType.DMA,
            pltpu.SemaphoreType.DMA,
        ],
    )
    def k(a_hbm, b_hbm, ei_hbm, g_hbm,
          idxd_all, idxs_all, a_v, b_v, o_v, a_sh, sem_a, sem_b):
        cc = lax.axis_index("c")
        ss = lax.axis_index("s")
        wid = ss * NC + cc
        l0 = wid * epw
        pltpu.sync_copy(ei_hbm.at[1, pl.ds(off + l0, epw)], idxd_all)
        pltpu.sync_copy(ei_hbm.at[0, pl.ds(off + l0, epw)], idxs_all)
        pltpu.sync_copy(a_hbm.at[pl.ds(ss * npt, npt)],
                        a_sh.at[pl.ds(ss * npt, npt)])
        plsc.subcore_barrier()
        pltpu.async_copy(a_sh.at[idxd_all.at[pl.ds(0, GK)]], a_v.at[0], sem_a)
        pltpu.async_copy(b_hbm.at[idxs_all.at[pl.ds(0, GK)]], b_v.at[0], sem_b)

        def step(j, carry):
            for p in range(2):
                g = 2 * j + p

                @pl.when(g < nch)
                def _():
                    pltpu.make_async_copy(
                        a_hbm.at[pl.ds(0, GK)], a_v.at[p], sem_a).wait()
                    pltpu.make_async_copy(
                        b_hbm.at[pl.ds(0, GK)], b_v.at[p], sem_b).wait()

                    @pl.when(g + 1 < nch)
                    def _():
                        o = (g + 1) * GK
                        pltpu.async_copy(a_sh.at[idxd_all.at[pl.ds(o, GK)]],
                                         a_v.at[1 - p], sem_a)
                        pltpu.async_copy(b_hbm.at[idxs_all.at[pl.ds(o, GK)]],
                                         b_v.at[1 - p], sem_b)

                    def row(rp, c2):
                        for half in range(2):
                            for t in range(H // 16):
                                sl = pl.ds(t * 16, 16)
                                ol = pl.ds(half * H + t * 16, 16)
                                o_v[p, rp, ol] = (a_v[p, 2 * rp + half, sl]
                                                  + b_v[p, 2 * rp + half, sl])
                        return c2

                    lax.fori_loop(0, GK // 2, row, 0)
                    pltpu.sync_copy(
                        o_v.at[p],
                        g_hbm.at[pl.ds((l0 + g * GK) // 2, GK // 2)])
            return carry

        lax.fori_loop(0, (nch + 1) // 2, step, 0)

    return k(A, B, edge_index)


def _tc_edge(g, e2, W1e, b1, W2, b2, pair_off):
    ne2 = g.shape[0]
    De2 = e2.shape[1]
    blk_off = pair_off // BEP
    W1e2 = jnp.kron(jnp.eye(2, dtype=jnp.float32), W1e)
    W2p = jnp.kron(jnp.eye(2, dtype=jnp.float32), W2)
    b1p = jnp.tile(b1, 2).reshape(1, 2 * H)
    b2p = jnp.tile(b2, 2).reshape(1, 2 * H)

    def body(g_ref, e_ref, w1_ref, b1_ref, w2_ref, b2_ref, m_ref):
        c = jnp.dot(e_ref[...], w1_ref[...], preferred_element_type=jnp.float32)
        p = jnp.maximum(g_ref[...] + c + b1_ref[...], 0.0)
        m = jnp.dot(p, w2_ref[...], preferred_element_type=jnp.float32)
        m_ref[...] = jnp.maximum(m + b2_ref[...], 0.0)

    return pl.pallas_call(
        body,
        grid=(ne2 // BEP,),
        in_specs=[
            pl.BlockSpec((BEP, 2 * H), lambda i: (i, 0)),
            pl.BlockSpec((BEP, De2), lambda i: (i + blk_off, 0)),
            pl.BlockSpec((De2, 2 * H), lambda i: (0, 0)),
            pl.BlockSpec((1, 2 * H), lambda i: (0, 0)),
            pl.BlockSpec((2 * H, 2 * H), lambda i: (0, 0)),
            pl.BlockSpec((1, 2 * H), lambda i: (0, 0)),
        ],
        out_specs=pl.BlockSpec((BEP, 2 * H), lambda i: (i, 0)),
        out_shape=jax.ShapeDtypeStruct((ne2, 2 * H), jnp.float32),
    )(g, e2, W1e2, b1p, W2p, b2p)


def _sc_scatter(M, edge_index, zeros_tile, N, off, ne):
    epw = ne // NW
    nch = epw // SK
    npt = N // NS
    mesh = plsc.VectorSubcoreMesh(core_axis_name="c", subcore_axis_name="s")

    @functools.partial(
        pl.kernel,
        out_type=jax.ShapeDtypeStruct((NC, N, H), jnp.float32),
        mesh=mesh,
        compiler_params=pltpu.CompilerParams(use_tc_tiling_on_sc=False),
        scratch_types=[
            pltpu.VMEM((2, SK), jnp.int32),
            pltpu.VMEM((2, SK // 2, 2 * H), jnp.float32),
            pltpu.VMEM((SK, H), jnp.float32),
            pltpu.VMEM_SHARED((N, H), jnp.float32),
            pltpu.SemaphoreType.DMA,
            pltpu.SemaphoreType.DMA,
        ],
    )
    def k(m_hbm, ei_hbm, z_hbm, out_hbm, idx_v, m_v, m64_v, acc_sh,
          sem_i, sem_m):
        c = lax.axis_index("c")
        s = lax.axis_index("s")
        wid = s * NC + c
        l0 = wid * epw
        pltpu.sync_copy(z_hbm, acc_sh.at[pl.ds(s * npt, npt)])
        plsc.subcore_barrier()
        pltpu.async_copy(ei_hbm.at[1, pl.ds(off + l0, SK)], idx_v.at[0],
                         sem_i)
        pltpu.async_copy(m_hbm.at[pl.ds(l0 // 2, SK // 2)], m_v.at[0], sem_m)

        def step(j, carry):
            for p in range(2):
                g = 2 * j + p

                @pl.when(g < nch)
                def _():
                    pltpu.make_async_copy(
                        ei_hbm.at[1, pl.ds(0, SK)], idx_v.at[p], sem_i).wait()
                    pltpu.make_async_copy(
                        m_hbm.at[pl.ds(0, SK // 2)], m_v.at[p], sem_m).wait()

                    @pl.when(g + 1 < nch)
                    def _():
                        o = l0 + (g + 1) * SK
                        pltpu.async_copy(ei_hbm.at[1, pl.ds(off + o, SK)],
                                         idx_v.at[1 - p], sem_i)
                        pltpu.async_copy(m_hbm.at[pl.ds(o // 2, SK // 2)],
                                         m_v.at[1 - p], sem_m)

                    def row(rp, c2):
                        for half in range(2):
                            for t in range(H // 16):
                                sl = pl.ds(half * H + t * 16, 16)
                                ol = pl.ds(t * 16, 16)
                                m64_v[2 * rp + half, ol] = m_v[p, rp, sl]
                        return c2

                    lax.fori_loop(0, SK // 2, row, 0)
                    pltpu.sync_copy(m64_v, acc_sh.at[idx_v.at[p]], add=True)
            return carry

        lax.fori_loop(0, (nch + 1) // 2, step, 0)
        plsc.subcore_barrier()
        pltpu.sync_copy(acc_sh.at[pl.ds(s * npt, npt)],
                        out_hbm.at[c, pl.ds(s * npt, npt)])

    return k(M, edge_index, zeros_tile)


def _tc_update(h, accs, W1h, W1a, b1, W2, b2, Wi, Wj):
    N = h.shape[0]

    def body(h_ref, p0_ref, p1_ref, p2_ref, p3_ref, w1h_ref, w1a_ref, b1_ref,
             w2_ref, b2_ref, wi_ref, wj_ref, h_out, a_out, b_out):
        aggr = ((p0_ref[...] + p1_ref[...]) + (p2_ref[...] + p3_ref[...]))
        u = jnp.dot(h_ref[...], w1h_ref[...], preferred_element_type=jnp.float32)
        u = u + jnp.dot(aggr, w1a_ref[...], preferred_element_type=jnp.float32)
        u = jnp.maximum(u + b1_ref[...], 0.0)
        u = jnp.dot(u, w2_ref[...], preferred_element_type=jnp.float32)
        u = jnp.maximum(u + b2_ref[...], 0.0)
        hn = u + h_ref[...]
        h_out[...] = hn
        a_out[...] = jnp.dot(hn, wi_ref[...], preferred_element_type=jnp.float32)
        b_out[...] = jnp.dot(hn, wj_ref[...], preferred_element_type=jnp.float32)

    return pl.pallas_call(
        body,
        out_shape=[jax.ShapeDtypeStruct((N, H), jnp.float32)] * 3,
    )(h, accs[0][0], accs[0][1], accs[1][0], accs[1][1], W1h, W1a,
      b1.reshape(1, H), W2, b2.reshape(1, H), Wi, Wj)


def _tc_final(h, accs, W1h, W1a, b1, W2, b2, tW1, tb1, tW2r, tb2):
    N = h.shape[0]
    Hh = tW1.shape[1]

    def body(h_ref, p0_ref, p1_ref, p2_ref, p3_ref, w1h_ref, w1a_ref, b1_ref,
             w2_ref, b2_ref, tw1_ref, tb1_ref, tw2_ref, tb2_ref,
             h_out, tok_out):
        aggr = ((p0_ref[...] + p1_ref[...]) + (p2_ref[...] + p3_ref[...]))
        u = jnp.dot(h_ref[...], w1h_ref[...], preferred_element_type=jnp.float32)
        u = u + jnp.dot(aggr, w1a_ref[...], preferred_element_type=jnp.float32)
        u = jnp.maximum(u + b1_ref[...], 0.0)
        u = jnp.dot(u, w2_ref[...], preferred_element_type=jnp.float32)
        u = jnp.maximum(u + b2_ref[...], 0.0)
        hn = u + h_ref[...]
        h_out[...] = hn
        t = jnp.dot(hn, tw1_ref[...], preferred_element_type=jnp.float32)
        t = jnp.maximum(t + tb1_ref[...], 0.0)
        tok_out[...] = jnp.sum(t * tw2_ref[...], axis=1) + tb2_ref[0, 0]

    return pl.pallas_call(
        body,
        out_shape=[jax.ShapeDtypeStruct((N, H), jnp.float32),
                   jax.ShapeDtypeStruct((N,), jnp.float32)],
    )(h, accs[0][0], accs[0][1], accs[1][0], accs[1][1], W1h, W1a,
      b1.reshape(1, H), W2, b2.reshape(1, H),
      tW1, tb1.reshape(1, Hh), tW2r, tb2.reshape(1, 1))


def kernel(x, edge_index, edge_attr, params):
    p = params
    N = x.shape[0]
    E = edge_index.shape[1]
    ne = E // NPART
    e2 = edge_attr.reshape(E // 2, 2 * edge_attr.shape[1])
    zeros_tile = jnp.zeros((N // NS, H), jnp.float32)

    h, A, B = _tc_pre(x, p['node_proj_W'], p['node_proj_b'],
                      p['l0_msg_W1'][:H], p['l0_msg_W1'][H:2 * H])
    num_layers = 3
    for i in range(num_layers):
        W1 = p[f'l{i}_msg_W1']
        W1e = p['edge_proj_W'] @ W1[2 * H:]
        b1e = p['edge_proj_b'] @ W1[2 * H:] + p[f'l{i}_msg_b1']
        gs = [None] * NPART
        Ms = [None] * NPART
        accs = [None] * NPART
        for k in range(NPART):
            gs[k] = _sc_gather(A, B, edge_index, k * ne, ne, N)
            if k > 0:
                Ms[k - 1] = _tc_edge(gs[k - 1], e2, W1e, b1e,
                                     p[f'l{i}_msg_W2'], p[f'l{i}_msg_b2'],
                                     (k - 1) * ne // 2)
        Ms[NPART - 1] = _tc_edge(gs[NPART - 1], e2, W1e, b1e,
                                 p[f'l{i}_msg_W2'], p[f'l{i}_msg_b2'],
                                 (NPART - 1) * ne // 2)
        for k in range(NPART):
            accs[k] = _sc_scatter(Ms[k], edge_index, zeros_tile, N, k * ne, ne)
        upW1 = p[f'l{i}_up_W1']
        if i < num_layers - 1:
            Wn1 = p[f'l{i + 1}_msg_W1']
            h, A, B = _tc_update(h, accs, upW1[:H], upW1[H:],
                                 p[f'l{i}_up_b1'], p[f'l{i}_up_W2'],
                                 p[f'l{i}_up_b2'], Wn1[:H], Wn1[H:2 * H])
        else:
            h, tok = _tc_final(h, accs, upW1[:H], upW1[H:],
                               p[f'l{i}_up_b1'], p[f'l{i}_up_W2'],
                               p[f'l{i}_up_b2'], p['tok_W1'], p['tok_b1'],
                               p['tok_W2'].reshape(1, H // 2), p['tok_b2'])
    return tok, h
